# Initial kernel scaffold; baseline (speedup 1.0000x reference)
import jax
import jax.numpy as jnp
from jax.experimental import pallas as pl


def kernel(numerical_data, cat_data, tables, W1, b1, W2, b2, Wo, bo, g0, be0, g1, be1, g2, be2):
    raise NotImplementedError("write your pallas kernel here")


# speedup vs baseline: 7.2384x; 7.2384x over previous
import functools

import jax
import jax.numpy as jnp
from jax import lax
from jax.experimental import pallas as pl
from jax.experimental.pallas import tpu as pltpu
from jax.experimental.pallas import tpu_sc as plsc

_EPS = 1e-5
_GATHER_WINDOW = 128


def _sc_gather(flat_table, idx):
    num_idx = idx.shape[0]
    d = flat_table.shape[1]
    idx2 = idx.reshape(1, num_idx)
    mesh = plsc.VectorSubcoreMesh(core_axis_name="core",
                                  subcore_axis_name="subcore")

    @functools.partial(
        pl.kernel,
        out_type=jax.ShapeDtypeStruct((num_idx, d), flat_table.dtype),
        mesh=mesh,
        compiler_params=pltpu.CompilerParams(use_tc_tiling_on_sc=False),
    )
    def k(x_hbm, i_hbm, o_hbm):
        def body(i_vmem, o_vmem):
            pltpu.sync_copy(x_hbm.at[i_vmem.at[0]], o_vmem)

        pltpu.emit_pipeline(
            body,
            grid=(num_idx // _GATHER_WINDOW,),
            in_specs=[pl.BlockSpec((1, _GATHER_WINDOW),
                                   index_map=lambda i: (0, i))],
            out_specs=[pl.BlockSpec((_GATHER_WINDOW, d),
                                    index_map=lambda i: (i, 0))],
            core_axis_name=("core", "subcore"),
            dimension_semantics=(pltpu.PARALLEL,),
        )(i_hbm, o_hbm)

    return k(flat_table, idx2)


def _a_body(emb_ref, numt_ref, num_ref, w1e_ref, w1n_ref, b1_ref,
            g0_ref, be0_ref, x1_ref, st1_ref):
    numt = numt_ref[...]
    nb = numt.shape[1]
    m0 = jnp.sum(numt, axis=1, keepdims=True) / nb
    v0 = jnp.sum(numt * numt, axis=1, keepdims=True) / nb - m0 * m0
    scale0 = g0_ref[...] * lax.rsqrt(v0 + _EPS)
    shift0 = be0_ref[...] - m0 * scale0
    w1n = w1n_ref[...]
    w1n_eff = w1n * scale0
    bias = b1_ref[...] + jnp.sum(w1n * shift0, axis=0, keepdims=True)

    x1 = jnp.dot(emb_ref[...], w1e_ref[...],
                 preferred_element_type=jnp.float32)
    x1 = x1 + jnp.dot(num_ref[...], w1n_eff,
                      preferred_element_type=jnp.float32)
    x1 = jnp.maximum(x1 + bias, 0.0)
    x1_ref[...] = x1

    @pl.when(pl.program_id(0) == 0)
    def _():
        st1_ref[...] = jnp.zeros_like(st1_ref)

    st1_ref[...] += jnp.concatenate(
        [jnp.sum(x1, axis=0, keepdims=True),
         jnp.sum(x1 * x1, axis=0, keepdims=True)], axis=0)


def _b_body(x1_ref, st1_ref, w2_ref, b2_ref, g1_ref, be1_ref,
            x2_ref, st2_ref, *, n_rows):
    m1 = st1_ref[0:1, :] / n_rows
    v1 = st1_ref[1:2, :] / n_rows - m1 * m1
    scale1 = g1_ref[...] * lax.rsqrt(v1 + _EPS)
    shift1 = be1_ref[...] - m1 * scale1
    xn = x1_ref[...] * scale1 + shift1
    x2 = jnp.dot(xn, w2_ref[...], preferred_element_type=jnp.float32)
    x2 = jnp.maximum(x2 + b2_ref[...], 0.0)
    x2_ref[...] = x2

    @pl.when(pl.program_id(0) == 0)
    def _():
        st2_ref[...] = jnp.zeros_like(st2_ref)

    st2_ref[...] += jnp.concatenate(
        [jnp.sum(x2, axis=0, keepdims=True),
         jnp.sum(x2 * x2, axis=0, keepdims=True)], axis=0)


def _c_body(x2_ref, st2_ref, wo_ref, g2_ref, be2_ref, bo_ref, o_ref,
            *, n_rows):
    m2 = st2_ref[0:1, :] / n_rows
    v2 = st2_ref[1:2, :] / n_rows - m2 * m2
    scale2 = g2_ref[...] * lax.rsqrt(v2 + _EPS)
    shift2 = be2_ref[...] - m2 * scale2
    xn = x2_ref[...] * scale2 + shift2
    o_ref[...] = jnp.dot(xn, wo_ref[...],
                         preferred_element_type=jnp.float32) + bo_ref[...]


def _mlp(emb, numerical_data, W1, b1, W2, b2, Wo, bo,
         g0, be0, g1, be1, g2, be2, tile):
    B, NUM = numerical_data.shape
    E = emb.shape[1]
    H1 = W1.shape[0]
    H2 = W2.shape[0]
    nb = B // tile

    x1, st1 = pl.pallas_call(
        _a_body,
        grid=(nb,),
        in_specs=[
            pl.BlockSpec((tile, E), lambda i: (i, 0)),
            pl.BlockSpec((NUM, B), lambda i: (0, 0)),
            pl.BlockSpec((tile, NUM), lambda i: (i, 0)),
            pl.BlockSpec((E, H1), lambda i: (0, 0)),
            pl.BlockSpec((NUM, H1), lambda i: (0, 0)),
            pl.BlockSpec((1, H1), lambda i: (0, 0)),
            pl.BlockSpec((NUM, 1), lambda i: (0, 0)),
            pl.BlockSpec((NUM, 1), lambda i: (0, 0)),
        ],
        out_specs=[
            pl.BlockSpec((tile, H1), lambda i: (i, 0)),
            pl.BlockSpec((2, H1), lambda i: (0, 0)),
        ],
        out_shape=[
            jax.ShapeDtypeStruct((B, H1), jnp.float32),
            jax.ShapeDtypeStruct((2, H1), jnp.float32),
        ],
    )(emb, numerical_data.T, numerical_data, W1[:, :E].T, W1[:, E:].T,
      b1[None, :], g0[:, None], be0[:, None])

    x2, st2 = pl.pallas_call(
        functools.partial(_b_body, n_rows=float(B)),
        grid=(nb,),
        in_specs=[
            pl.BlockSpec((tile, H1), lambda i: (i, 0)),
            pl.BlockSpec((2, H1), lambda i: (0, 0)),
            pl.BlockSpec((H1, H2), lambda i: (0, 0)),
            pl.BlockSpec((1, H2), lambda i: (0, 0)),
            pl.BlockSpec((1, H1), lambda i: (0, 0)),
            pl.BlockSpec((1, H1), lambda i: (0, 0)),
        ],
        out_specs=[
            pl.BlockSpec((tile, H2), lambda i: (i, 0)),
            pl.BlockSpec((2, H2), lambda i: (0, 0)),
        ],
        out_shape=[
            jax.ShapeDtypeStruct((B, H2), jnp.float32),
            jax.ShapeDtypeStruct((2, H2), jnp.float32),
        ],
    )(x1, st1, W2.T, b2[None, :], g1[None, :], be1[None, :])

    out = pl.pallas_call(
        functools.partial(_c_body, n_rows=float(B)),
        grid=(nb,),
        in_specs=[
            pl.BlockSpec((tile, H2), lambda i: (i, 0)),
            pl.BlockSpec((2, H2), lambda i: (0, 0)),
            pl.BlockSpec((H2, 1), lambda i: (0, 0)),
            pl.BlockSpec((1, H2), lambda i: (0, 0)),
            pl.BlockSpec((1, H2), lambda i: (0, 0)),
            pl.BlockSpec((1, 1), lambda i: (0, 0)),
        ],
        out_specs=pl.BlockSpec((tile, 1), lambda i: (i, 0)),
        out_shape=jax.ShapeDtypeStruct((B, 1), jnp.float32),
    )(x2, st2, Wo.T, g2[None, :], be2[None, :], bo[None, :])

    return out


def kernel(numerical_data, cat_data, tables, W1, b1, W2, b2, Wo, bo,
           g0, be0, g1, be1, g2, be2):
    B, NUM = numerical_data.shape
    F, V, D = tables.shape
    flat_table = tables.reshape(F * V, D)
    offs = (jnp.arange(F, dtype=jnp.int32) * V)[None, :]
    idx = (cat_data.astype(jnp.int32) + offs).reshape(B * F)
    rows = _sc_gather(flat_table, idx)
    emb = rows.reshape(B, F * D)
    return _mlp(emb, numerical_data, W1, b1, W2, b2, Wo, bo,
                g0, be0, g1, be1, g2, be2, tile=2048)


# speedup vs baseline: 29.8089x; 4.1182x over previous
import functools

import jax
import jax.numpy as jnp
from jax import lax
from jax.experimental import pallas as pl
from jax.experimental.pallas import tpu as pltpu
from jax.experimental.pallas import tpu_sc as plsc

_EPS = 1e-5


def _sc_gather_t(m, cat_t, f_per_row):
    R, V = m.shape
    F, B = cat_t.shape
    info = plsc.get_sparse_core_info()
    nw = info.num_cores * info.num_subcores
    rows_per_w = R // nw
    out_chunk = 2048
    mesh = plsc.VectorSubcoreMesh(core_axis_name="core",
                                  subcore_axis_name="subcore")

    @functools.partial(
        pl.kernel,
        out_type=jax.ShapeDtypeStruct((R, B), jnp.float32),
        mesh=mesh,
        compiler_params=pltpu.CompilerParams(needs_layout_passes=False),
        scratch_types=[
            pltpu.VMEM((V,), jnp.float32),
            pltpu.VMEM((B,), jnp.int32),
            pltpu.VMEM((out_chunk,), jnp.float32),
        ],
    )
    def k(m_hbm, cat_hbm, out_hbm, row_v, idx_v, out_v):
        wid = (lax.axis_index("subcore") * info.num_cores
               + lax.axis_index("core"))

        @pl.loop(0, rows_per_w)
        def _(j):
            r = wid * rows_per_w + j
            f = r // f_per_row
            pltpu.sync_copy(cat_hbm.at[f], idx_v)
            pltpu.sync_copy(m_hbm.at[r], row_v)
            for c in range(B // out_chunk):
                @pl.loop(0, out_chunk // 16)
                def _(t):
                    iv = idx_v[pl.ds(c * out_chunk + t * 16, 16)]
                    out_v[pl.ds(t * 16, 16)] = plsc.load_gather(row_v, [iv])
                pltpu.sync_copy(out_v, out_hbm.at[r, pl.ds(c * out_chunk,
                                                           out_chunk)])

    return k(m, cat_t)


def _a_body(embt_ref, numt_ref, num_ref, w1e_ref, w1n_ref, b1_ref,
            g0_ref, be0_ref, x1_ref, st1_ref):
    numt = numt_ref[...]
    nb = numt.shape[1]
    m0 = jnp.sum(numt, axis=1, keepdims=True) / nb
    v0 = jnp.sum(numt * numt, axis=1, keepdims=True) / nb - m0 * m0
    scale0 = g0_ref[...] * lax.rsqrt(v0 + _EPS)
    shift0 = be0_ref[...] - m0 * scale0
    w1n = w1n_ref[...]
    w1n_eff = w1n * scale0
    bias = b1_ref[...] + jnp.sum(w1n * shift0, axis=0, keepdims=True)

    x1 = lax.dot_general(embt_ref[...], w1e_ref[...],
                         (((0,), (0,)), ((), ())),
                         preferred_element_type=jnp.float32)
    x1 = x1 + jnp.dot(num_ref[...], w1n_eff,
                      preferred_element_type=jnp.float32)
    x1 = jnp.maximum(x1 + bias, 0.0)
    x1_ref[...] = x1

    @pl.when(pl.program_id(0) == 0)
    def _():
        st1_ref[...] = jnp.zeros_like(st1_ref)

    st1_ref[...] += jnp.concatenate(
        [jnp.sum(x1, axis=0, keepdims=True),
         jnp.sum(x1 * x1, axis=0, keepdims=True)], axis=0)


def _b_body(x1_ref, st1_ref, w2_ref, b2_ref, g1_ref, be1_ref,
            x2_ref, st2_ref, *, n_rows):
    m1 = st1_ref[0:1, :] / n_rows
    v1 = st1_ref[1:2, :] / n_rows - m1 * m1
    scale1 = g1_ref[...] * lax.rsqrt(v1 + _EPS)
    shift1 = be1_ref[...] - m1 * scale1
    xn = x1_ref[...] * scale1 + shift1
    x2 = jnp.dot(xn, w2_ref[...], preferred_element_type=jnp.float32)
    x2 = jnp.maximum(x2 + b2_ref[...], 0.0)
    x2_ref[...] = x2

    @pl.when(pl.program_id(0) == 0)
    def _():
        st2_ref[...] = jnp.zeros_like(st2_ref)

    st2_ref[...] += jnp.concatenate(
        [jnp.sum(x2, axis=0, keepdims=True),
         jnp.sum(x2 * x2, axis=0, keepdims=True)], axis=0)


def _c_body(x2_ref, st2_ref, wo_ref, g2_ref, be2_ref, bo_ref, o_ref,
            *, n_rows):
    m2 = st2_ref[0:1, :] / n_rows
    v2 = st2_ref[1:2, :] / n_rows - m2 * m2
    scale2 = g2_ref[...] * lax.rsqrt(v2 + _EPS)
    shift2 = be2_ref[...] - m2 * scale2
    xn = x2_ref[...] * scale2 + shift2
    o_ref[...] = jnp.dot(xn, wo_ref[...],
                         preferred_element_type=jnp.float32) + bo_ref[...]


def _mlp(embt, numerical_data, W1, b1, W2, b2, Wo, bo,
         g0, be0, g1, be1, g2, be2, tile):
    B, NUM = numerical_data.shape
    E = embt.shape[0]
    H1 = W1.shape[0]
    H2 = W2.shape[0]
    nb = B // tile

    x1, st1 = pl.pallas_call(
        _a_body,
        grid=(nb,),
        in_specs=[
            pl.BlockSpec((E, tile), lambda i: (0, i)),
            pl.BlockSpec((NUM, B), lambda i: (0, 0)),
            pl.BlockSpec((tile, NUM), lambda i: (i, 0)),
            pl.BlockSpec((E, H1), lambda i: (0, 0)),
            pl.BlockSpec((NUM, H1), lambda i: (0, 0)),
            pl.BlockSpec((1, H1), lambda i: (0, 0)),
            pl.BlockSpec((NUM, 1), lambda i: (0, 0)),
            pl.BlockSpec((NUM, 1), lambda i: (0, 0)),
        ],
        out_specs=[
            pl.BlockSpec((tile, H1), lambda i: (i, 0)),
            pl.BlockSpec((2, H1), lambda i: (0, 0)),
        ],
        out_shape=[
            jax.ShapeDtypeStruct((B, H1), jnp.float32),
            jax.ShapeDtypeStruct((2, H1), jnp.float32),
        ],
    )(embt, numerical_data.T, numerical_data, W1[:, :E].T, W1[:, E:].T,
      b1[None, :], g0[:, None], be0[:, None])

    x2, st2 = pl.pallas_call(
        functools.partial(_b_body, n_rows=float(B)),
        grid=(nb,),
        in_specs=[
            pl.BlockSpec((tile, H1), lambda i: (i, 0)),
            pl.BlockSpec((2, H1), lambda i: (0, 0)),
            pl.BlockSpec((H1, H2), lambda i: (0, 0)),
            pl.BlockSpec((1, H2), lambda i: (0, 0)),
            pl.BlockSpec((1, H1), lambda i: (0, 0)),
            pl.BlockSpec((1, H1), lambda i: (0, 0)),
        ],
        out_specs=[
            pl.BlockSpec((tile, H2), lambda i: (i, 0)),
            pl.BlockSpec((2, H2), lambda i: (0, 0)),
        ],
        out_shape=[
            jax.ShapeDtypeStruct((B, H2), jnp.float32),
            jax.ShapeDtypeStruct((2, H2), jnp.float32),
        ],
    )(x1, st1, W2.T, b2[None, :], g1[None, :], be1[None, :])

    out = pl.pallas_call(
        functools.partial(_c_body, n_rows=float(B)),
        grid=(nb,),
        in_specs=[
            pl.BlockSpec((tile, H2), lambda i: (i, 0)),
            pl.BlockSpec((2, H2), lambda i: (0, 0)),
            pl.BlockSpec((H2, 1), lambda i: (0, 0)),
            pl.BlockSpec((1, H2), lambda i: (0, 0)),
            pl.BlockSpec((1, H2), lambda i: (0, 0)),
            pl.BlockSpec((1, 1), lambda i: (0, 0)),
        ],
        out_specs=pl.BlockSpec((tile, 1), lambda i: (i, 0)),
        out_shape=jax.ShapeDtypeStruct((B, 1), jnp.float32),
    )(x2, st2, Wo.T, g2[None, :], be2[None, :], bo[None, :])

    return out


def kernel(numerical_data, cat_data, tables, W1, b1, W2, b2, Wo, bo,
           g0, be0, g1, be1, g2, be2):
    B, NUM = numerical_data.shape
    F, V, D = tables.shape
    m = tables.transpose(0, 2, 1).reshape(F * D, V)
    cat_t = cat_data.T.astype(jnp.int32)
    embt = _sc_gather_t(m, cat_t, f_per_row=D)
    return _mlp(embt, numerical_data, W1, b1, W2, b2, Wo, bo,
                g0, be0, g1, be1, g2, be2, tile=2048)


# speedup vs baseline: 42.1277x; 1.4133x over previous
import functools

import jax
import jax.numpy as jnp
from jax import lax
from jax.experimental import pallas as pl
from jax.experimental.pallas import tpu as pltpu
from jax.experimental.pallas import tpu_sc as plsc

_EPS = 1e-5


def _sc_gather_t(m, cat_t, f_per_row):
    R, V = m.shape
    F, B = cat_t.shape
    info = plsc.get_sparse_core_info()
    nw = info.num_cores * info.num_subcores
    rows_per_w = R // nw
    out_chunk = 4096
    n_chunks = B // out_chunk
    unroll = 8
    mesh = plsc.VectorSubcoreMesh(core_axis_name="core",
                                  subcore_axis_name="subcore")

    @functools.partial(
        pl.kernel,
        out_type=jax.ShapeDtypeStruct((R, B), jnp.float32),
        mesh=mesh,
        compiler_params=pltpu.CompilerParams(needs_layout_passes=False),
        scratch_types=[
            pltpu.VMEM((V,), jnp.float32),
            pltpu.VMEM((B,), jnp.int32),
            pltpu.VMEM((out_chunk,), jnp.float32),
            pltpu.VMEM((out_chunk,), jnp.float32),
            pltpu.SemaphoreType.DMA,
            pltpu.SemaphoreType.DMA,
            pltpu.SemaphoreType.DMA,
        ],
    )
    def k(m_hbm, cat_hbm, out_hbm, row_v, idx_v, ob0, ob1, sem_row,
          sem_o0, sem_o1):
        wid = (lax.axis_index("subcore") * info.num_cores
               + lax.axis_index("core"))
        row0 = wid * rows_per_w
        obufs = (ob0, ob1)
        osems = (sem_o0, sem_o1)

        pltpu.make_async_copy(m_hbm.at[row0], row_v, sem_row).start()

        @pl.loop(0, rows_per_w)
        def _(j):
            r = row0 + j

            @pl.when(jnp.logical_or(j == 0, lax.rem(r, f_per_row) == 0))
            def _():
                pltpu.sync_copy(cat_hbm.at[r // f_per_row], idx_v)

            pltpu.make_async_copy(m_hbm.at[r], row_v, sem_row).wait()

            for c in range(n_chunks):
                ob = obufs[c % 2]
                osem = osems[c % 2]
                if c >= 2:
                    pltpu.make_async_copy(
                        ob, out_hbm.at[0, pl.ds(0, out_chunk)], osem).wait()
                else:
                    @pl.when(j > 0)
                    def _():
                        pltpu.make_async_copy(
                            ob, out_hbm.at[0, pl.ds(0, out_chunk)],
                            osem).wait()

                @pl.loop(0, out_chunk // 16, step=unroll)
                def _(t):
                    for u in range(unroll):
                        iv = idx_v[pl.ds(c * out_chunk + (t + u) * 16, 16)]
                        ob[pl.ds((t + u) * 16, 16)] = plsc.load_gather(
                            row_v, [iv])

                if c == n_chunks - 1:
                    @pl.when(j + 1 < rows_per_w)
                    def _():
                        pltpu.make_async_copy(m_hbm.at[r + 1], row_v,
                                              sem_row).start()
                pltpu.make_async_copy(
                    ob, out_hbm.at[r, pl.ds(c * out_chunk, out_chunk)],
                    osem).start()

        for p in range(2):
            pltpu.make_async_copy(obufs[p],
                                  out_hbm.at[0, pl.ds(0, out_chunk)],
                                  osems[p]).wait()

    return k(m, cat_t)


def _a_body(embt_ref, numt_ref, num_ref, w1e_ref, w1n_ref, b1_ref,
            g0_ref, be0_ref, x1_ref, st1_ref):
    numt = numt_ref[...]
    nb = numt.shape[1]
    m0 = jnp.sum(numt, axis=1, keepdims=True) / nb
    v0 = jnp.sum(numt * numt, axis=1, keepdims=True) / nb - m0 * m0
    scale0 = g0_ref[...] * lax.rsqrt(v0 + _EPS)
    shift0 = be0_ref[...] - m0 * scale0
    w1n = w1n_ref[...]
    w1n_eff = w1n * scale0
    bias = b1_ref[...] + jnp.sum(w1n * shift0, axis=0, keepdims=True)

    x1 = lax.dot_general(embt_ref[...], w1e_ref[...],
                         (((0,), (0,)), ((), ())),
                         preferred_element_type=jnp.float32)
    x1 = x1 + jnp.dot(num_ref[...], w1n_eff,
                      preferred_element_type=jnp.float32)
    x1 = jnp.maximum(x1 + bias, 0.0)
    x1_ref[...] = x1

    @pl.when(pl.program_id(0) == 0)
    def _():
        st1_ref[...] = jnp.zeros_like(st1_ref)

    st1_ref[...] += jnp.concatenate(
        [jnp.sum(x1, axis=0, keepdims=True),
         jnp.sum(x1 * x1, axis=0, keepdims=True)], axis=0)


def _b_body(x1_ref, st1_ref, w2_ref, b2_ref, g1_ref, be1_ref,
            x2_ref, st2_ref, *, n_rows):
    m1 = st1_ref[0:1, :] / n_rows
    v1 = st1_ref[1:2, :] / n_rows - m1 * m1
    scale1 = g1_ref[...] * lax.rsqrt(v1 + _EPS)
    shift1 = be1_ref[...] - m1 * scale1
    xn = x1_ref[...] * scale1 + shift1
    x2 = jnp.dot(xn, w2_ref[...], preferred_element_type=jnp.float32)
    x2 = jnp.maximum(x2 + b2_ref[...], 0.0)
    x2_ref[...] = x2

    @pl.when(pl.program_id(0) == 0)
    def _():
        st2_ref[...] = jnp.zeros_like(st2_ref)

    st2_ref[...] += jnp.concatenate(
        [jnp.sum(x2, axis=0, keepdims=True),
         jnp.sum(x2 * x2, axis=0, keepdims=True)], axis=0)


def _c_body(x2_ref, st2_ref, wo_ref, g2_ref, be2_ref, bo_ref, o_ref,
            *, n_rows):
    m2 = st2_ref[0:1, :] / n_rows
    v2 = st2_ref[1:2, :] / n_rows - m2 * m2
    scale2 = g2_ref[...] * lax.rsqrt(v2 + _EPS)
    shift2 = be2_ref[...] - m2 * scale2
    xn = x2_ref[...] * scale2 + shift2
    o_ref[...] = jnp.dot(xn, wo_ref[...],
                         preferred_element_type=jnp.float32) + bo_ref[...]


def _mlp(embt, numerical_data, W1, b1, W2, b2, Wo, bo,
         g0, be0, g1, be1, g2, be2, tile):
    B, NUM = numerical_data.shape
    E = embt.shape[0]
    H1 = W1.shape[0]
    H2 = W2.shape[0]
    nb = B // tile

    x1, st1 = pl.pallas_call(
        _a_body,
        grid=(nb,),
        in_specs=[
            pl.BlockSpec((E, tile), lambda i: (0, i)),
            pl.BlockSpec((NUM, B), lambda i: (0, 0)),
            pl.BlockSpec((tile, NUM), lambda i: (i, 0)),
            pl.BlockSpec((E, H1), lambda i: (0, 0)),
            pl.BlockSpec((NUM, H1), lambda i: (0, 0)),
            pl.BlockSpec((1, H1), lambda i: (0, 0)),
            pl.BlockSpec((NUM, 1), lambda i: (0, 0)),
            pl.BlockSpec((NUM, 1), lambda i: (0, 0)),
        ],
        out_specs=[
            pl.BlockSpec((tile, H1), lambda i: (i, 0)),
            pl.BlockSpec((2, H1), lambda i: (0, 0)),
        ],
        out_shape=[
            jax.ShapeDtypeStruct((B, H1), jnp.float32),
            jax.ShapeDtypeStruct((2, H1), jnp.float32),
        ],
    )(embt, numerical_data.T, numerical_data, W1[:, :E].T, W1[:, E:].T,
      b1[None, :], g0[:, None], be0[:, None])

    x2, st2 = pl.pallas_call(
        functools.partial(_b_body, n_rows=float(B)),
        grid=(nb,),
        in_specs=[
            pl.BlockSpec((tile, H1), lambda i: (i, 0)),
            pl.BlockSpec((2, H1), lambda i: (0, 0)),
            pl.BlockSpec((H1, H2), lambda i: (0, 0)),
            pl.BlockSpec((1, H2), lambda i: (0, 0)),
            pl.BlockSpec((1, H1), lambda i: (0, 0)),
            pl.BlockSpec((1, H1), lambda i: (0, 0)),
        ],
        out_specs=[
            pl.BlockSpec((tile, H2), lambda i: (i, 0)),
            pl.BlockSpec((2, H2), lambda i: (0, 0)),
        ],
        out_shape=[
            jax.ShapeDtypeStruct((B, H2), jnp.float32),
            jax.ShapeDtypeStruct((2, H2), jnp.float32),
        ],
    )(x1, st1, W2.T, b2[None, :], g1[None, :], be1[None, :])

    out = pl.pallas_call(
        functools.partial(_c_body, n_rows=float(B)),
        grid=(nb,),
        in_specs=[
            pl.BlockSpec((tile, H2), lambda i: (i, 0)),
            pl.BlockSpec((2, H2), lambda i: (0, 0)),
            pl.BlockSpec((H2, 1), lambda i: (0, 0)),
            pl.BlockSpec((1, H2), lambda i: (0, 0)),
            pl.BlockSpec((1, H2), lambda i: (0, 0)),
            pl.BlockSpec((1, 1), lambda i: (0, 0)),
        ],
        out_specs=pl.BlockSpec((tile, 1), lambda i: (i, 0)),
        out_shape=jax.ShapeDtypeStruct((B, 1), jnp.float32),
    )(x2, st2, Wo.T, g2[None, :], be2[None, :], bo[None, :])

    return out


def kernel(numerical_data, cat_data, tables, W1, b1, W2, b2, Wo, bo,
           g0, be0, g1, be1, g2, be2):
    B, NUM = numerical_data.shape
    F, V, D = tables.shape
    m = tables.transpose(0, 2, 1).reshape(F * D, V)
    cat_t = cat_data.T.astype(jnp.int32)
    embt = _sc_gather_t(m, cat_t, f_per_row=D)
    return _mlp(embt, numerical_data, W1, b1, W2, b2, Wo, bo,
                g0, be0, g1, be1, g2, be2, tile=2048)


# speedup vs baseline: 45.1108x; 1.0708x over previous
import functools

import jax
import jax.numpy as jnp
from jax import lax
from jax.experimental import pallas as pl
from jax.experimental.pallas import tpu as pltpu
from jax.experimental.pallas import tpu_sc as plsc

_EPS = 1e-5


def _sc_gather_t(m, cat_t, f_per_row):
    R, V = m.shape
    F, B = cat_t.shape
    info = plsc.get_sparse_core_info()
    nw = info.num_cores * info.num_subcores
    rows_per_w = R // nw
    out_chunk = 4096
    n_chunks = B // out_chunk
    unroll = 8
    mesh = plsc.VectorSubcoreMesh(core_axis_name="core",
                                  subcore_axis_name="subcore")

    @functools.partial(
        pl.kernel,
        out_type=jax.ShapeDtypeStruct((R, B), jnp.float32),
        mesh=mesh,
        compiler_params=pltpu.CompilerParams(needs_layout_passes=False),
        scratch_types=[
            pltpu.VMEM((V,), jnp.float32),
            pltpu.VMEM((B,), jnp.int32),
            pltpu.VMEM((out_chunk,), jnp.float32),
            pltpu.VMEM((out_chunk,), jnp.float32),
            pltpu.SemaphoreType.DMA,
            pltpu.SemaphoreType.DMA,
            pltpu.SemaphoreType.DMA,
        ],
    )
    def k(m_hbm, cat_hbm, out_hbm, row_v, idx_v, ob0, ob1, sem_row,
          sem_o0, sem_o1):
        wid = (lax.axis_index("subcore") * info.num_cores
               + lax.axis_index("core"))
        row0 = wid * rows_per_w
        obufs = (ob0, ob1)
        osems = (sem_o0, sem_o1)

        pltpu.make_async_copy(m_hbm.at[row0], row_v, sem_row).start()

        @pl.loop(0, rows_per_w)
        def _(j):
            r = row0 + j

            @pl.when(jnp.logical_or(j == 0, lax.rem(r, f_per_row) == 0))
            def _():
                pltpu.sync_copy(cat_hbm.at[r // f_per_row], idx_v)

            pltpu.make_async_copy(m_hbm.at[r], row_v, sem_row).wait()

            for c in range(n_chunks):
                ob = obufs[c % 2]
                osem = osems[c % 2]
                if c >= 2:
                    pltpu.make_async_copy(
                        ob, out_hbm.at[0, pl.ds(0, out_chunk)], osem).wait()
                else:
                    @pl.when(j > 0)
                    def _():
                        pltpu.make_async_copy(
                            ob, out_hbm.at[0, pl.ds(0, out_chunk)],
                            osem).wait()

                @pl.loop(0, out_chunk // 16, step=unroll)
                def _(t):
                    for u in range(unroll):
                        iv = idx_v[pl.ds(c * out_chunk + (t + u) * 16, 16)]
                        ob[pl.ds((t + u) * 16, 16)] = plsc.load_gather(
                            row_v, [iv])

                if c == n_chunks - 1:
                    @pl.when(j + 1 < rows_per_w)
                    def _():
                        pltpu.make_async_copy(m_hbm.at[r + 1], row_v,
                                              sem_row).start()
                pltpu.make_async_copy(
                    ob, out_hbm.at[r, pl.ds(c * out_chunk, out_chunk)],
                    osem).start()

        for p in range(2):
            pltpu.make_async_copy(obufs[p],
                                  out_hbm.at[0, pl.ds(0, out_chunk)],
                                  osems[p]).wait()

    return k(m, cat_t)


def _fused_body(embt_ref, numt_ref, num_ref, w1e_ref, w1n_ref, b1_ref,
                g0_ref, be0_ref, w2_ref, b2_ref, g1_ref, be1_ref,
                wo_ref, g2_ref, be2_ref, bo_ref, o_ref,
                x1_s, x2_s, st1_s, st2_s, *, tile, n_rows):
    p = pl.program_id(0)
    i = pl.program_id(1)
    rows = pl.ds(i * tile, tile)

    @pl.when(p == 0)
    def _():
        numt = numt_ref[...]
        m0 = jnp.sum(numt, axis=1, keepdims=True) / n_rows
        v0 = jnp.sum(numt * numt, axis=1, keepdims=True) / n_rows - m0 * m0
        scale0 = g0_ref[...] * lax.rsqrt(v0 + _EPS)
        shift0 = be0_ref[...] - m0 * scale0
        w1n = w1n_ref[...]
        bias = b1_ref[...] + jnp.sum(w1n * shift0, axis=0, keepdims=True)

        x1 = lax.dot_general(embt_ref[...], w1e_ref[...],
                             (((0,), (0,)), ((), ())),
                             preferred_element_type=jnp.float32)
        x1 = x1 + jnp.dot(num_ref[...], w1n * scale0,
                          preferred_element_type=jnp.float32)
        x1 = jnp.maximum(x1 + bias, 0.0)

        @pl.when(i == 0)
        def _():
            st1_s[...] = jnp.zeros_like(st1_s)

        st1_s[...] += jnp.concatenate(
            [jnp.sum(x1, axis=0, keepdims=True),
             jnp.sum(x1 * x1, axis=0, keepdims=True)], axis=0)
        x1_s[rows, :] = x1.astype(jnp.bfloat16)

    @pl.when(p == 1)
    def _():
        m1 = st1_s[0:1, :] / n_rows
        v1 = st1_s[1:2, :] / n_rows - m1 * m1
        scale1 = g1_ref[...] * lax.rsqrt(v1 + _EPS)
        shift1 = be1_ref[...] - m1 * scale1
        xn = x1_s[rows, :].astype(jnp.float32) * scale1 + shift1
        x2 = jnp.dot(xn, w2_ref[...], preferred_element_type=jnp.float32)
        x2 = jnp.maximum(x2 + b2_ref[...], 0.0)

        @pl.when(i == 0)
        def _():
            st2_s[...] = jnp.zeros_like(st2_s)

        st2_s[...] += jnp.concatenate(
            [jnp.sum(x2, axis=0, keepdims=True),
             jnp.sum(x2 * x2, axis=0, keepdims=True)], axis=0)
        x2_s[rows, :] = x2

    @pl.when(p == 2)
    def _():
        m2 = st2_s[0:1, :] / n_rows
        v2 = st2_s[1:2, :] / n_rows - m2 * m2
        scale2 = g2_ref[...] * lax.rsqrt(v2 + _EPS)
        shift2 = be2_ref[...] - m2 * scale2
        xn = x2_s[rows, :] * scale2 + shift2
        o_ref[...] = jnp.dot(xn, wo_ref[...],
                             preferred_element_type=jnp.float32) + bo_ref[...]


def _mlp(embt, numerical_data, W1, b1, W2, b2, Wo, bo,
         g0, be0, g1, be1, g2, be2, tile):
    B, NUM = numerical_data.shape
    E = embt.shape[0]
    H1 = W1.shape[0]
    H2 = W2.shape[0]
    nb = B // tile
    const = lambda p, i: (0, 0)

    return pl.pallas_call(
        functools.partial(_fused_body, tile=tile, n_rows=float(B)),
        grid=(3, nb),
        in_specs=[
            pl.BlockSpec((E, tile), lambda p, i: (0, jnp.where(p == 0, i, 0))),
            pl.BlockSpec((NUM, B), const),
            pl.BlockSpec((tile, NUM),
                         lambda p, i: (jnp.where(p == 0, i, 0), 0)),
            pl.BlockSpec((E, H1), const),
            pl.BlockSpec((NUM, H1), const),
            pl.BlockSpec((1, H1), const),
            pl.BlockSpec((NUM, 1), const),
            pl.BlockSpec((NUM, 1), const),
            pl.BlockSpec((H1, H2), const),
            pl.BlockSpec((1, H2), const),
            pl.BlockSpec((1, H1), const),
            pl.BlockSpec((1, H1), const),
            pl.BlockSpec((H2, 1), const),
            pl.BlockSpec((1, H2), const),
            pl.BlockSpec((1, H2), const),
            pl.BlockSpec((1, 1), const),
        ],
        out_specs=pl.BlockSpec((tile, 1),
                               lambda p, i: (jnp.where(p == 2, i, 0), 0)),
        out_shape=jax.ShapeDtypeStruct((B, 1), jnp.float32),
        scratch_shapes=[
            pltpu.VMEM((B, H1), jnp.bfloat16),
            pltpu.VMEM((B, H2), jnp.float32),
            pltpu.VMEM((2, H1), jnp.float32),
            pltpu.VMEM((2, H2), jnp.float32),
        ],
    )(embt, numerical_data.T, numerical_data, W1[:, :E].T, W1[:, E:].T,
      b1[None, :], g0[:, None], be0[:, None], W2.T, b2[None, :],
      g1[None, :], be1[None, :], Wo.T, g2[None, :], be2[None, :],
      bo[None, :])


def kernel(numerical_data, cat_data, tables, W1, b1, W2, b2, Wo, bo,
           g0, be0, g1, be1, g2, be2):
    B, NUM = numerical_data.shape
    F, V, D = tables.shape
    m = tables.transpose(0, 2, 1).reshape(F * D, V)
    cat_t = cat_data.T.astype(jnp.int32)
    embt = _sc_gather_t(m, cat_t, f_per_row=D)
    return _mlp(embt, numerical_data, W1, b1, W2, b2, Wo, bo,
                g0, be0, g1, be1, g2, be2, tile=2048)


# speedup vs baseline: 45.1495x; 1.0009x over previous
import functools

import jax
import jax.numpy as jnp
from jax import lax
from jax.experimental import pallas as pl
from jax.experimental.pallas import tpu as pltpu
from jax.experimental.pallas import tpu_sc as plsc

_EPS = 1e-5


def _sc_gather_t(m, cat_t, f_per_row):
    R, V = m.shape
    F, B = cat_t.shape
    info = plsc.get_sparse_core_info()
    nw = info.num_cores * info.num_subcores
    rows_per_w = R // nw
    out_chunk = 4096
    n_chunks = B // out_chunk
    unroll = 8
    mesh = plsc.VectorSubcoreMesh(core_axis_name="core",
                                  subcore_axis_name="subcore")

    @functools.partial(
        pl.kernel,
        out_type=jax.ShapeDtypeStruct((R, B), jnp.float32),
        mesh=mesh,
        compiler_params=pltpu.CompilerParams(needs_layout_passes=False),
        scratch_types=[
            pltpu.VMEM((V,), jnp.float32),
            pltpu.VMEM((B,), jnp.int32),
            pltpu.VMEM((out_chunk,), jnp.float32),
            pltpu.VMEM((out_chunk,), jnp.float32),
            pltpu.SemaphoreType.DMA,
            pltpu.SemaphoreType.DMA,
            pltpu.SemaphoreType.DMA,
        ],
    )
    def k(m_hbm, cat_hbm, out_hbm, row_v, idx_v, ob0, ob1, sem_row,
          sem_o0, sem_o1):
        wid = (lax.axis_index("subcore") * info.num_cores
               + lax.axis_index("core"))
        row0 = wid * rows_per_w
        obufs = (ob0, ob1)
        osems = (sem_o0, sem_o1)

        pltpu.make_async_copy(m_hbm.at[row0], row_v, sem_row).start()

        @pl.loop(0, rows_per_w)
        def _(j):
            r = row0 + j

            @pl.when(jnp.logical_or(j == 0, lax.rem(r, f_per_row) == 0))
            def _():
                pltpu.sync_copy(cat_hbm.at[r // f_per_row], idx_v)

            pltpu.make_async_copy(m_hbm.at[r], row_v, sem_row).wait()

            for c in range(n_chunks):
                ob = obufs[c % 2]
                osem = osems[c % 2]
                if c >= 2:
                    pltpu.make_async_copy(
                        ob, out_hbm.at[0, pl.ds(0, out_chunk)], osem).wait()
                else:
                    @pl.when(j > 0)
                    def _():
                        pltpu.make_async_copy(
                            ob, out_hbm.at[0, pl.ds(0, out_chunk)],
                            osem).wait()

                @pl.loop(0, out_chunk // 16, step=unroll)
                def _(t):
                    for u in range(unroll):
                        iv = idx_v[pl.ds(c * out_chunk + (t + u) * 16, 16)]
                        ob[pl.ds((t + u) * 16, 16)] = plsc.load_gather(
                            row_v, [iv])

                if c == n_chunks - 1:
                    @pl.when(j + 1 < rows_per_w)
                    def _():
                        pltpu.make_async_copy(m_hbm.at[r + 1], row_v,
                                              sem_row).start()
                pltpu.make_async_copy(
                    ob, out_hbm.at[r, pl.ds(c * out_chunk, out_chunk)],
                    osem).start()

        for p in range(2):
            pltpu.make_async_copy(obufs[p],
                                  out_hbm.at[0, pl.ds(0, out_chunk)],
                                  osems[p]).wait()

    return k(m, cat_t)


def _fused_body(embt_ref, numt_ref, num_ref, w1e_ref, w1n_ref, b1_ref,
                g0_ref, be0_ref, w2_ref, b2_ref, g1_ref, be1_ref,
                wo_ref, g2_ref, be2_ref, bo_ref, o_ref,
                x1_s, x2_s, st1_s, st2_s, *, tile, n_rows):
    p = pl.program_id(0)
    i = pl.program_id(1)
    rows = pl.ds(i * tile, tile)

    @pl.when(p == 0)
    def _():
        numt = numt_ref[...]
        m0 = jnp.sum(numt, axis=1, keepdims=True) / n_rows
        v0 = jnp.sum(numt * numt, axis=1, keepdims=True) / n_rows - m0 * m0
        scale0 = g0_ref[...] * lax.rsqrt(v0 + _EPS)
        shift0 = be0_ref[...] - m0 * scale0
        w1n = w1n_ref[...]
        bias = b1_ref[...] + jnp.sum(w1n * shift0, axis=0, keepdims=True)

        x1 = lax.dot_general(embt_ref[...].astype(jnp.bfloat16),
                             w1e_ref[...].astype(jnp.bfloat16),
                             (((0,), (0,)), ((), ())),
                             preferred_element_type=jnp.float32)
        x1 = x1 + jnp.dot(num_ref[...], w1n * scale0,
                          preferred_element_type=jnp.float32)
        x1 = jnp.maximum(x1 + bias, 0.0)

        @pl.when(i == 0)
        def _():
            st1_s[...] = jnp.zeros_like(st1_s)

        st1_s[...] += jnp.concatenate(
            [jnp.sum(x1, axis=0, keepdims=True),
             jnp.sum(x1 * x1, axis=0, keepdims=True)], axis=0)
        x1_s[rows, :] = x1.astype(jnp.bfloat16)

    @pl.when(p == 1)
    def _():
        m1 = st1_s[0:1, :] / n_rows
        v1 = st1_s[1:2, :] / n_rows - m1 * m1
        scale1 = g1_ref[...] * lax.rsqrt(v1 + _EPS)
        shift1 = be1_ref[...] - m1 * scale1
        xn = x1_s[rows, :].astype(jnp.float32) * scale1 + shift1
        x2 = jnp.dot(xn.astype(jnp.bfloat16),
                     w2_ref[...].astype(jnp.bfloat16),
                     preferred_element_type=jnp.float32)
        x2 = jnp.maximum(x2 + b2_ref[...], 0.0)

        @pl.when(i == 0)
        def _():
            st2_s[...] = jnp.zeros_like(st2_s)

        st2_s[...] += jnp.concatenate(
            [jnp.sum(x2, axis=0, keepdims=True),
             jnp.sum(x2 * x2, axis=0, keepdims=True)], axis=0)
        x2_s[rows, :] = x2

    @pl.when(p == 2)
    def _():
        m2 = st2_s[0:1, :] / n_rows
        v2 = st2_s[1:2, :] / n_rows - m2 * m2
        scale2 = g2_ref[...] * lax.rsqrt(v2 + _EPS)
        shift2 = be2_ref[...] - m2 * scale2
        xn = x2_s[rows, :] * scale2 + shift2
        o_ref[...] = jnp.dot(xn, wo_ref[...],
                             preferred_element_type=jnp.float32) + bo_ref[...]


def _mlp(embt, numerical_data, W1, b1, W2, b2, Wo, bo,
         g0, be0, g1, be1, g2, be2, tile):
    B, NUM = numerical_data.shape
    E = embt.shape[0]
    H1 = W1.shape[0]
    H2 = W2.shape[0]
    nb = B // tile
    const = lambda p, i: (0, 0)

    return pl.pallas_call(
        functools.partial(_fused_body, tile=tile, n_rows=float(B)),
        grid=(3, nb),
        in_specs=[
            pl.BlockSpec((E, tile), lambda p, i: (0, jnp.where(p == 0, i, 0))),
            pl.BlockSpec((NUM, B), const),
            pl.BlockSpec((tile, NUM),
                         lambda p, i: (jnp.where(p == 0, i, 0), 0)),
            pl.BlockSpec((E, H1), const),
            pl.BlockSpec((NUM, H1), const),
            pl.BlockSpec((1, H1), const),
            pl.BlockSpec((NUM, 1), const),
            pl.BlockSpec((NUM, 1), const),
            pl.BlockSpec((H1, H2), const),
            pl.BlockSpec((1, H2), const),
            pl.BlockSpec((1, H1), const),
            pl.BlockSpec((1, H1), const),
            pl.BlockSpec((H2, 1), const),
            pl.BlockSpec((1, H2), const),
            pl.BlockSpec((1, H2), const),
            pl.BlockSpec((1, 1), const),
        ],
        out_specs=pl.BlockSpec((tile, 1),
                               lambda p, i: (jnp.where(p == 2, i, 0), 0)),
        out_shape=jax.ShapeDtypeStruct((B, 1), jnp.float32),
        scratch_shapes=[
            pltpu.VMEM((B, H1), jnp.bfloat16),
            pltpu.VMEM((B, H2), jnp.float32),
            pltpu.VMEM((2, H1), jnp.float32),
            pltpu.VMEM((2, H2), jnp.float32),
        ],
    )(embt, numerical_data.T, numerical_data, W1[:, :E].T, W1[:, E:].T,
      b1[None, :], g0[:, None], be0[:, None], W2.T, b2[None, :],
      g1[None, :], be1[None, :], Wo.T, g2[None, :], be2[None, :],
      bo[None, :])


def kernel(numerical_data, cat_data, tables, W1, b1, W2, b2, Wo, bo,
           g0, be0, g1, be1, g2, be2):
    B, NUM = numerical_data.shape
    F, V, D = tables.shape
    m = tables.transpose(0, 2, 1).reshape(F * D, V)
    cat_t = cat_data.T.astype(jnp.int32)
    embt = _sc_gather_t(m, cat_t, f_per_row=D)
    return _mlp(embt, numerical_data, W1, b1, W2, b2, Wo, bo,
                g0, be0, g1, be1, g2, be2, tile=2048)
